# Initial kernel scaffold; baseline (speedup 1.0000x reference)
import jax
import jax.numpy as jnp
from jax.experimental import pallas as pl


def kernel(feat, edge_index, edge_attr, W_lin, b_lin, W_res, b_res):
    raise NotImplementedError("write your pallas kernel here")


# speedup vs baseline: 3.4766x; 3.4766x over previous
import functools

import jax
import jax.numpy as jnp
from jax import lax
from jax.experimental import pallas as pl
from jax.experimental.pallas import tpu as pltpu
from jax.experimental.pallas import tpu_sc as plsc

NC = 2
NS = 16
NW = NC * NS
LANES = 16
IDX_W = 128
CHUNK = 512
EB = 2048


def _node_proj_kernel(f_ref, w_ref, b_ref, p_ref, r_ref):
    o = jnp.dot(f_ref[...], w_ref[...], preferred_element_type=jnp.float32)
    o = o + b_ref[...]
    d = p_ref.shape[1]
    p_ref[...] = o[:, :d]
    r_ref[...] = o[:, d:]


def _edge_proj_kernel(ea_ref, w_ref, o_ref):
    o_ref[...] = jnp.dot(ea_ref[...], w_ref[...],
                         preferred_element_type=jnp.float32)


def _final_kernel(p_ref, r_ref, o_ref):
    n = o_ref.shape[0]
    o_ref[...] = p_ref[0, :n] + p_ref[1, :n] + r_ref[...]


def _make_sc_kernel(n_nodes, d_out, n_chunks, acc_rows):
    mesh = plsc.VectorSubcoreMesh(core_axis_name="c", subcore_axis_name="s")
    stripe = acc_rows // NS
    streams = CHUNK // IDX_W

    @functools.partial(
        pl.kernel,
        out_type=jax.ShapeDtypeStruct((NC, acc_rows, d_out), jnp.float32),
        mesh=mesh,
        compiler_params=pltpu.CompilerParams(use_tc_tiling_on_sc=False),
        scratch_types=[
            pltpu.VMEM((2, streams, IDX_W), jnp.int32),
            pltpu.VMEM((3, streams, IDX_W), jnp.int32),
            pltpu.VMEM((2, CHUNK, d_out), jnp.float32),
            pltpu.VMEM((2, CHUNK, d_out), jnp.float32),
            pltpu.VMEM_SHARED((acc_rows, d_out), jnp.float32),
            pltpu.SemaphoreType.DMA,
            pltpu.SemaphoreType.DMA,
            pltpu.SemaphoreType.DMA,
            pltpu.SemaphoreType.DMA,
            pltpu.SemaphoreType.DMA,
        ],
    )
    def sc_fn(proj_hbm, eproj_hbm, src_hbm, dst_hbm, zeros_hbm, out_hbm,
              idx_s, idx_d, rows, ep, acc, semz, semin0, semin1, gsem, ssem):
        cid = lax.axis_index("c")
        sid = lax.axis_index("s")
        wid = sid * NC + cid
        semin = (semin0, semin1)
        pltpu.async_copy(zeros_hbm.at[pl.ds(sid * stripe, stripe)],
                         acc.at[pl.ds(sid * stripe, stripe)], semz).wait()
        plsc.subcore_barrier()

        pend_in = [None, None]
        pend_g = [None, None]
        pend_s = [None, None]

        def issue_in(k):
            b = k % 2
            g = wid * n_chunks + k
            pend_in[b] = [
                pltpu.async_copy(src_hbm.at[pl.ds(g * streams, streams)],
                                 idx_s.at[b], semin[b]),
                pltpu.async_copy(dst_hbm.at[pl.ds(g * streams, streams)],
                                 idx_d.at[k % 3], semin[b]),
                pltpu.async_copy(eproj_hbm.at[pl.ds(g * CHUNK, CHUNK)],
                                 ep.at[b], semin[b]),
            ]

        def issue_gathers(k):
            b = k % 2
            pend_g[b] = [
                pltpu.async_copy(proj_hbm.at[idx_s.at[b, j]],
                                 rows.at[b, pl.ds(j * IDX_W, IDX_W)], gsem)
                for j in range(streams)]

        def issue_scatters(k):
            b = k % 2
            pend_s[b] = [
                pltpu.async_copy(rows.at[b, pl.ds(j * IDX_W, IDX_W)],
                                 acc.at[idx_d.at[k % 3, j]], ssem, add=True)
                for j in range(streams)]

        issue_in(0)
        for cp in pend_in[0]:
            cp.wait()
        issue_gathers(0)
        for k in range(n_chunks):
            b = k % 2
            nb = b ^ 1
            if k >= 1:
                for cp in pend_s[nb]:
                    cp.wait()
            if k + 1 < n_chunks:
                issue_in(k + 1)
                for cp in pend_in[nb]:
                    cp.wait()
                issue_gathers(k + 1)
            for cp in pend_g[b]:
                cp.wait()

            @pl.loop(0, CHUNK, unroll=8)
            def _(r):
                for c0 in range(0, d_out, LANES):
                    slc = (b, pl.ds(r, 1), pl.ds(c0, LANES))
                    rows.at[slc][...] = jnp.maximum(
                        rows.at[slc][...] + ep.at[slc][...], 0.0)

            issue_scatters(k)
        for cp in pend_s[(n_chunks - 1) % 2]:
            cp.wait()
        plsc.subcore_barrier()
        pltpu.sync_copy(acc.at[pl.ds(sid * stripe, stripe)],
                        out_hbm.at[cid, pl.ds(sid * stripe, stripe)])

    return sc_fn


def kernel(feat, edge_index, edge_attr, W_lin, b_lin, W_res, b_res):
    n_nodes, d_feat = feat.shape
    n_edges = edge_index.shape[1]
    d_out = W_res.shape[1]
    d_edge = edge_attr.shape[1]

    w_cat = jnp.concatenate([W_lin[:d_feat], W_res], axis=1)
    b_cat = jnp.concatenate([b_lin, b_res]).reshape(1, -1)
    proj, res = pl.pallas_call(
        _node_proj_kernel,
        out_shape=(jax.ShapeDtypeStruct((n_nodes, d_out), jnp.float32),
                   jax.ShapeDtypeStruct((n_nodes, d_out), jnp.float32)),
    )(feat, w_cat, b_cat)

    per_pass = NW * CHUNK
    n_chunks = -(-n_edges // per_pass)
    e_pad = n_chunks * per_pass
    eproj = pl.pallas_call(
        _edge_proj_kernel,
        grid=(-(-n_edges // EB),),
        in_specs=[pl.BlockSpec((EB, d_edge), lambda i: (i, 0)),
                  pl.BlockSpec((d_edge, d_out), lambda i: (0, 0))],
        out_specs=pl.BlockSpec((EB, d_out), lambda i: (i, 0)),
        out_shape=jax.ShapeDtypeStruct((e_pad, d_out), jnp.float32),
    )(edge_attr, W_lin[d_feat:])

    pad = e_pad - n_edges
    src_p = jnp.concatenate(
        [edge_index[0].astype(jnp.int32), jnp.zeros((pad,), jnp.int32)]
    ).reshape(-1, IDX_W)
    dst_p = jnp.concatenate(
        [edge_index[1].astype(jnp.int32),
         jnp.full((pad,), n_nodes, jnp.int32)]
    ).reshape(-1, IDX_W)

    acc_rows = -(-(n_nodes + 1) // (NS * 8)) * (NS * 8)
    zeros = jnp.zeros((acc_rows, d_out), jnp.float32)
    parts = _make_sc_kernel(n_nodes, d_out, n_chunks, acc_rows)(
        proj, eproj, src_p, dst_p, zeros)

    return pl.pallas_call(
        _final_kernel,
        out_shape=jax.ShapeDtypeStruct((n_nodes, d_out), jnp.float32),
    )(parts, res)


# speedup vs baseline: 5.4135x; 1.5571x over previous
import functools

import jax
import jax.numpy as jnp
from jax import lax
from jax.experimental import pallas as pl
from jax.experimental.pallas import tpu as pltpu
from jax.experimental.pallas import tpu_sc as plsc

NC = 2
NS = 16
NW = NC * NS
LANES = 16
IDX_W = 128
CHUNK = 512
EB = 2048


def _node_proj_kernel(f_ref, w_ref, b_ref, p_ref, r_ref):
    o = jnp.dot(f_ref[...], w_ref[...], preferred_element_type=jnp.float32)
    o = o + b_ref[...]
    d = p_ref.shape[1]
    p_ref[...] = o[:, :d]
    r_ref[...] = o[:, d:]


def _edge_proj_kernel(ea_ref, w_ref, o_ref):
    o_ref[...] = jnp.dot(ea_ref[...], w_ref[...],
                         preferred_element_type=jnp.float32)


def _final_kernel(p_ref, r_ref, o_ref):
    n = o_ref.shape[0]
    o_ref[...] = p_ref[0, :n] + p_ref[1, :n] + r_ref[...]


def _make_sc_kernel(n_nodes, d_out, n_chunks, acc_rows):
    mesh = plsc.VectorSubcoreMesh(core_axis_name="c", subcore_axis_name="s")
    stripe = acc_rows // NS
    streams = CHUNK // IDX_W

    @functools.partial(
        pl.kernel,
        out_type=jax.ShapeDtypeStruct((NC, acc_rows, d_out), jnp.float32),
        mesh=mesh,
        compiler_params=pltpu.CompilerParams(use_tc_tiling_on_sc=False),
        scratch_types=[
            pltpu.VMEM((2, streams, IDX_W), jnp.int32),
            pltpu.VMEM((3, streams, IDX_W), jnp.int32),
            pltpu.VMEM((2, CHUNK, d_out), jnp.float32),
            pltpu.VMEM((2, CHUNK, d_out), jnp.float32),
            pltpu.VMEM_SHARED((acc_rows, d_out), jnp.float32),
            pltpu.SemaphoreType.DMA,
            pltpu.SemaphoreType.DMA,
            pltpu.SemaphoreType.DMA,
            pltpu.SemaphoreType.DMA,
            pltpu.SemaphoreType.DMA,
        ],
    )
    def sc_fn(proj_hbm, eproj_hbm, src_hbm, dst_hbm, zeros_hbm, out_hbm,
              idx_s, idx_d, rows, ep, acc, semz, semin0, semin1, gsem, ssem):
        cid = lax.axis_index("c")
        sid = lax.axis_index("s")
        wid = sid * NC + cid
        semin = (semin0, semin1)
        pltpu.async_copy(zeros_hbm.at[pl.ds(sid * stripe, stripe)],
                         acc.at[pl.ds(sid * stripe, stripe)], semz).wait()
        plsc.subcore_barrier()

        pend_in = [None, None]
        pend_g = [None, None]
        pend_s = [None, None]

        def issue_in(k):
            b = k % 2
            g = wid * n_chunks + k
            pend_in[b] = [
                pltpu.async_copy(src_hbm.at[pl.ds(g * streams, streams)],
                                 idx_s.at[b], semin[b]),
                pltpu.async_copy(dst_hbm.at[pl.ds(g * streams, streams)],
                                 idx_d.at[k % 3], semin[b]),
                pltpu.async_copy(eproj_hbm.at[pl.ds(g * CHUNK, CHUNK)],
                                 ep.at[b], semin[b]),
            ]

        def issue_gathers(k):
            b = k % 2
            pend_g[b] = [
                pltpu.async_copy(proj_hbm.at[idx_s.at[b, j]],
                                 rows.at[b, pl.ds(j * IDX_W, IDX_W)], gsem)
                for j in range(streams)]

        def issue_scatters(k):
            b = k % 2
            pend_s[b] = [
                pltpu.async_copy(rows.at[b, pl.ds(j * IDX_W, IDX_W)],
                                 acc.at[idx_d.at[k % 3, j]], ssem, add=True)
                for j in range(streams)]

        issue_in(0)
        for cp in pend_in[0]:
            cp.wait()
        issue_gathers(0)
        for k in range(n_chunks):
            b = k % 2
            nb = b ^ 1
            if k >= 1:
                for cp in pend_s[nb]:
                    cp.wait()
            if k + 1 < n_chunks:
                issue_in(k + 1)
                for cp in pend_in[nb]:
                    cp.wait()
                issue_gathers(k + 1)
            for cp in pend_g[b]:
                cp.wait()

            @plsc.parallel_loop(0, CHUNK, unroll=8)
            def _(r):
                for c0 in range(0, d_out, LANES):
                    slc = (b, pl.ds(r, 1), pl.ds(c0, LANES))
                    rows.at[slc][...] = jnp.maximum(
                        rows.at[slc][...] + ep.at[slc][...], 0.0)

            issue_scatters(k)
        for cp in pend_s[(n_chunks - 1) % 2]:
            cp.wait()
        plsc.subcore_barrier()
        pltpu.sync_copy(acc.at[pl.ds(sid * stripe, stripe)],
                        out_hbm.at[cid, pl.ds(sid * stripe, stripe)])

    return sc_fn


def kernel(feat, edge_index, edge_attr, W_lin, b_lin, W_res, b_res):
    n_nodes, d_feat = feat.shape
    n_edges = edge_index.shape[1]
    d_out = W_res.shape[1]
    d_edge = edge_attr.shape[1]

    w_cat = jnp.concatenate([W_lin[:d_feat], W_res], axis=1)
    b_cat = jnp.concatenate([b_lin, b_res]).reshape(1, -1)
    proj, res = pl.pallas_call(
        _node_proj_kernel,
        out_shape=(jax.ShapeDtypeStruct((n_nodes, d_out), jnp.float32),
                   jax.ShapeDtypeStruct((n_nodes, d_out), jnp.float32)),
    )(feat, w_cat, b_cat)

    per_pass = NW * CHUNK
    n_chunks = -(-n_edges // per_pass)
    e_pad = n_chunks * per_pass
    real8 = n_edges // 8
    w2bd = jnp.kron(jnp.eye(8, dtype=jnp.float32), W_lin[d_feat:])
    eproj = pl.pallas_call(
        _edge_proj_kernel,
        grid=(-(-real8 // EB),),
        in_specs=[pl.BlockSpec((EB, 8 * d_edge), lambda i: (i, 0)),
                  pl.BlockSpec((8 * d_edge, 8 * d_out), lambda i: (0, 0))],
        out_specs=pl.BlockSpec((EB, 8 * d_out), lambda i: (i, 0)),
        out_shape=jax.ShapeDtypeStruct((e_pad // 8, 8 * d_out), jnp.float32),
    )(edge_attr.reshape(real8, 8 * d_edge), w2bd).reshape(e_pad, d_out)

    pad = e_pad - n_edges
    src_p = jnp.concatenate(
        [edge_index[0].astype(jnp.int32), jnp.zeros((pad,), jnp.int32)]
    ).reshape(-1, IDX_W)
    dst_p = jnp.concatenate(
        [edge_index[1].astype(jnp.int32),
         jnp.full((pad,), n_nodes, jnp.int32)]
    ).reshape(-1, IDX_W)

    acc_rows = -(-(n_nodes + 1) // (NS * 8)) * (NS * 8)
    zeros = jnp.zeros((acc_rows, d_out), jnp.float32)
    parts = _make_sc_kernel(n_nodes, d_out, n_chunks, acc_rows)(
        proj, eproj, src_p, dst_p, zeros)

    return pl.pallas_call(
        _final_kernel,
        out_shape=jax.ShapeDtypeStruct((n_nodes, d_out), jnp.float32),
    )(parts, res)


# speedup vs baseline: 6.3572x; 1.1743x over previous
import functools

import jax
import jax.numpy as jnp
from jax import lax
from jax.experimental import pallas as pl
from jax.experimental.pallas import tpu as pltpu
from jax.experimental.pallas import tpu_sc as plsc

NC = 2
NS = 16
NW = NC * NS
LANES = 16
IDX_W = 128
CHUNK = 512
EB = 2048


def _node_proj_kernel(f_ref, w_ref, b_ref, p_ref, r_ref):
    o = jnp.dot(f_ref[...], w_ref[...], preferred_element_type=jnp.float32)
    o = o + b_ref[...]
    n, d = r_ref.shape
    p_ref[pl.ds(0, n)] = o[:, :d]
    r_ref[...] = o[:, d:]


def _edge_proj_kernel(ea_ref, w_ref, o_ref):
    o_ref[...] = jnp.dot(ea_ref[...], w_ref[...],
                         preferred_element_type=jnp.float32)


def _final_kernel(p_ref, r_ref, o_ref):
    n = o_ref.shape[0]
    o_ref[...] = p_ref[0, :n] + p_ref[1, :n] + r_ref[...]


def _make_sc_kernel(n_nodes, d_out, n_chunks, acc_rows):
    mesh = plsc.VectorSubcoreMesh(core_axis_name="c", subcore_axis_name="s")
    stripe = acc_rows // NS
    streams = CHUNK // IDX_W

    @functools.partial(
        pl.kernel,
        out_type=jax.ShapeDtypeStruct((NC, acc_rows, d_out), jnp.float32),
        mesh=mesh,
        compiler_params=pltpu.CompilerParams(use_tc_tiling_on_sc=False),
        scratch_types=[
            pltpu.VMEM((2, streams, IDX_W), jnp.int32),
            pltpu.VMEM((3, streams, IDX_W), jnp.int32),
            pltpu.VMEM((2, CHUNK, d_out), jnp.float32),
            pltpu.VMEM((2, CHUNK, d_out), jnp.float32),
            pltpu.VMEM_SHARED((acc_rows, d_out), jnp.float32),
            pltpu.VMEM_SHARED((acc_rows, d_out), jnp.float32),
            pltpu.SemaphoreType.DMA,
            pltpu.SemaphoreType.DMA,
            pltpu.SemaphoreType.DMA,
            pltpu.SemaphoreType.DMA,
            pltpu.SemaphoreType.DMA,
        ],
    )
    def sc_fn(proj_hbm, eproj_hbm, src_hbm, dst_hbm, zeros_hbm, out_hbm,
              idx_s, idx_d, rows, ep, acc, table, semz, semin0, semin1,
              gsem, ssem):
        cid = lax.axis_index("c")
        sid = lax.axis_index("s")
        wid = sid * NC + cid
        semin = (semin0, semin1)
        stg = pltpu.async_copy(proj_hbm.at[pl.ds(sid * stripe, stripe)],
                               table.at[pl.ds(sid * stripe, stripe)], semz)
        pltpu.async_copy(zeros_hbm.at[pl.ds(sid * stripe, stripe)],
                         acc.at[pl.ds(sid * stripe, stripe)], semz).wait()
        stg.wait()
        plsc.subcore_barrier()

        pend_in = [None, None]
        pend_g = [None, None]
        pend_s = [None, None]

        def issue_in(k):
            b = k % 2
            g = wid * n_chunks + k
            pend_in[b] = [
                pltpu.async_copy(src_hbm.at[pl.ds(g * streams, streams)],
                                 idx_s.at[b], semin[b]),
                pltpu.async_copy(dst_hbm.at[pl.ds(g * streams, streams)],
                                 idx_d.at[k % 3], semin[b]),
                pltpu.async_copy(eproj_hbm.at[pl.ds(g * CHUNK, CHUNK)],
                                 ep.at[b], semin[b]),
            ]

        def issue_gathers(k):
            b = k % 2
            pend_g[b] = [
                pltpu.async_copy(table.at[idx_s.at[b, j]],
                                 rows.at[b, pl.ds(j * IDX_W, IDX_W)], gsem)
                for j in range(streams)]

        def issue_scatters(k):
            b = k % 2
            pend_s[b] = [
                pltpu.async_copy(rows.at[b, pl.ds(j * IDX_W, IDX_W)],
                                 acc.at[idx_d.at[k % 3, j]], ssem, add=True)
                for j in range(streams)]

        issue_in(0)
        for cp in pend_in[0]:
            cp.wait()
        issue_gathers(0)
        for k in range(n_chunks):
            b = k % 2
            nb = b ^ 1
            if k >= 1:
                for cp in pend_s[nb]:
                    cp.wait()
            if k + 1 < n_chunks:
                issue_in(k + 1)
                for cp in pend_in[nb]:
                    cp.wait()
                issue_gathers(k + 1)
            for cp in pend_g[b]:
                cp.wait()

            @plsc.parallel_loop(0, CHUNK, unroll=8)
            def _(r):
                for c0 in range(0, d_out, LANES):
                    slc = (b, pl.ds(r, 1), pl.ds(c0, LANES))
                    rows.at[slc][...] = jnp.maximum(
                        rows.at[slc][...] + ep.at[slc][...], 0.0)

            issue_scatters(k)
        for cp in pend_s[(n_chunks - 1) % 2]:
            cp.wait()
        plsc.subcore_barrier()
        pltpu.sync_copy(acc.at[pl.ds(sid * stripe, stripe)],
                        out_hbm.at[cid, pl.ds(sid * stripe, stripe)])

    return sc_fn


def kernel(feat, edge_index, edge_attr, W_lin, b_lin, W_res, b_res):
    n_nodes, d_feat = feat.shape
    n_edges = edge_index.shape[1]
    d_out = W_res.shape[1]
    d_edge = edge_attr.shape[1]

    acc_rows = -(-(n_nodes + 1) // (NS * 8)) * (NS * 8)
    w_cat = jnp.concatenate([W_lin[:d_feat], W_res], axis=1)
    b_cat = jnp.concatenate([b_lin, b_res]).reshape(1, -1)
    proj, res = pl.pallas_call(
        _node_proj_kernel,
        out_shape=(jax.ShapeDtypeStruct((acc_rows, d_out), jnp.float32),
                   jax.ShapeDtypeStruct((n_nodes, d_out), jnp.float32)),
    )(feat, w_cat, b_cat)

    per_pass = NW * CHUNK
    n_chunks = -(-n_edges // per_pass)
    e_pad = n_chunks * per_pass
    real8 = n_edges // 8
    w2bd = jnp.kron(jnp.eye(8, dtype=jnp.float32), W_lin[d_feat:])
    eproj = pl.pallas_call(
        _edge_proj_kernel,
        grid=(-(-real8 // EB),),
        in_specs=[pl.BlockSpec((EB, 8 * d_edge), lambda i: (i, 0)),
                  pl.BlockSpec((8 * d_edge, 8 * d_out), lambda i: (0, 0))],
        out_specs=pl.BlockSpec((EB, 8 * d_out), lambda i: (i, 0)),
        out_shape=jax.ShapeDtypeStruct((e_pad // 8, 8 * d_out), jnp.float32),
    )(edge_attr.reshape(real8, 8 * d_edge), w2bd).reshape(e_pad, d_out)

    pad = e_pad - n_edges
    src_p = jnp.concatenate(
        [edge_index[0].astype(jnp.int32), jnp.zeros((pad,), jnp.int32)]
    ).reshape(-1, IDX_W)
    dst_p = jnp.concatenate(
        [edge_index[1].astype(jnp.int32),
         jnp.full((pad,), n_nodes, jnp.int32)]
    ).reshape(-1, IDX_W)

    zeros = jnp.zeros((acc_rows, d_out), jnp.float32)
    parts = _make_sc_kernel(n_nodes, d_out, n_chunks, acc_rows)(
        proj, eproj, src_p, dst_p, zeros)

    return pl.pallas_call(
        _final_kernel,
        out_shape=jax.ShapeDtypeStruct((n_nodes, d_out), jnp.float32),
    )(parts, res)


# speedup vs baseline: 6.4146x; 1.0090x over previous
import functools

import jax
import jax.numpy as jnp
from jax import lax
from jax.experimental import pallas as pl
from jax.experimental.pallas import tpu as pltpu
from jax.experimental.pallas import tpu_sc as plsc

NC = 2
NS = 16
NW = NC * NS
LANES = 16
IDX_W = 128
CHUNK = 512
EB = 2048


def _node_proj_kernel(f_ref, w_ref, b_ref, p_ref, r_ref):
    o = jnp.dot(f_ref[...], w_ref[...], preferred_element_type=jnp.float32)
    o = o + b_ref[...]
    n, d = r_ref.shape
    p_ref[pl.ds(0, n)] = o[:, :d]
    r_ref[...] = o[:, d:]


def _edge_proj_kernel(ea_ref, w_ref, o_ref):
    o_ref[...] = jnp.dot(ea_ref[...], w_ref[...],
                         preferred_element_type=jnp.float32)


def _final_kernel(p_ref, r_ref, o_ref):
    n = o_ref.shape[0]
    o_ref[...] = p_ref[0, :n] + p_ref[1, :n] + r_ref[...]


def _make_sc_kernel(n_nodes, d_out, n_chunks, total_chunks, acc_rows):
    mesh = plsc.VectorSubcoreMesh(core_axis_name="c", subcore_axis_name="s")
    stripe = acc_rows // NS
    streams = CHUNK // IDX_W

    @functools.partial(
        pl.kernel,
        out_type=jax.ShapeDtypeStruct((NC, acc_rows, d_out), jnp.float32),
        mesh=mesh,
        compiler_params=pltpu.CompilerParams(use_tc_tiling_on_sc=False),
        scratch_types=[
            pltpu.VMEM((2, streams, IDX_W), jnp.int32),
            pltpu.VMEM((3, streams, IDX_W), jnp.int32),
            pltpu.VMEM((2, CHUNK, d_out), jnp.float32),
            pltpu.VMEM((2, CHUNK, d_out), jnp.float32),
            pltpu.VMEM_SHARED((acc_rows, d_out), jnp.float32),
            pltpu.VMEM_SHARED((acc_rows, d_out), jnp.float32),
            pltpu.SemaphoreType.DMA,
            pltpu.SemaphoreType.DMA,
            pltpu.SemaphoreType.DMA,
            pltpu.SemaphoreType.DMA,
            pltpu.SemaphoreType.DMA,
        ],
    )
    def sc_fn(proj_hbm, eproj_hbm, eidx_hbm, zeros_hbm, out_hbm,
              idx_s, idx_d, rows, ep, acc, table, semz, semin0, semin1,
              gsem, ssem):
        cid = lax.axis_index("c")
        sid = lax.axis_index("s")
        wid = sid * NC + cid
        semin = (semin0, semin1)
        stg = pltpu.async_copy(proj_hbm.at[pl.ds(sid * stripe, stripe)],
                               table.at[pl.ds(sid * stripe, stripe)], semz)
        pltpu.async_copy(zeros_hbm.at[pl.ds(sid * stripe, stripe)],
                         acc.at[pl.ds(sid * stripe, stripe)], semz).wait()
        stg.wait()
        plsc.subcore_barrier()

        pend_in = [None, None]
        pend_g = [None, None]
        pend_s = [None, None]

        def chunk_id(k):
            g = wid + NW * k
            if (k + 1) * NW > total_chunks:
                g = jnp.minimum(g, total_chunks - 1)
            return g

        def issue_in(k):
            b = k % 2
            g = chunk_id(k)
            pend_in[b] = [
                pltpu.async_copy(eidx_hbm.at[0, pl.ds(g * streams, streams)],
                                 idx_s.at[b], semin[b]),
                pltpu.async_copy(eidx_hbm.at[1, pl.ds(g * streams, streams)],
                                 idx_d.at[k % 3], semin[b]),
                pltpu.async_copy(eproj_hbm.at[pl.ds(g * CHUNK, CHUNK)],
                                 ep.at[b], semin[b]),
            ]

        def issue_gathers(k):
            b = k % 2
            pend_g[b] = [
                pltpu.async_copy(table.at[idx_s.at[b, j]],
                                 rows.at[b, pl.ds(j * IDX_W, IDX_W)], gsem)
                for j in range(streams)]

        def issue_scatters(k):
            b = k % 2
            pend_s[b] = [
                pltpu.async_copy(rows.at[b, pl.ds(j * IDX_W, IDX_W)],
                                 acc.at[idx_d.at[k % 3, j]], ssem, add=True)
                for j in range(streams)]

        issue_in(0)
        for cp in pend_in[0]:
            cp.wait()
        issue_gathers(0)
        for k in range(n_chunks):
            b = k % 2
            nb = b ^ 1
            if k >= 1:
                for cp in pend_s[nb]:
                    cp.wait()
            if k + 1 < n_chunks:
                issue_in(k + 1)
                for cp in pend_in[nb]:
                    cp.wait()
                if (k + 2) * NW > total_chunks:
                    ok = (wid + NW * (k + 1)) < total_chunks
                    for j in range(streams):
                        for c0 in range(0, IDX_W, LANES):
                            slc = ((k + 1) % 3, j, pl.ds(c0, LANES))
                            idx_d.at[slc][...] = jnp.where(
                                ok, idx_d.at[slc][...], n_nodes)
                issue_gathers(k + 1)
            for cp in pend_g[b]:
                cp.wait()

            @plsc.parallel_loop(0, CHUNK, unroll=8)
            def _(r):
                for c0 in range(0, d_out, LANES):
                    slc = (b, pl.ds(r, 1), pl.ds(c0, LANES))
                    rows.at[slc][...] = jnp.maximum(
                        rows.at[slc][...] + ep.at[slc][...], 0.0)

            issue_scatters(k)
        for cp in pend_s[(n_chunks - 1) % 2]:
            cp.wait()
        plsc.subcore_barrier()
        pltpu.sync_copy(acc.at[pl.ds(sid * stripe, stripe)],
                        out_hbm.at[cid, pl.ds(sid * stripe, stripe)])

    return sc_fn


def kernel(feat, edge_index, edge_attr, W_lin, b_lin, W_res, b_res):
    n_nodes, d_feat = feat.shape
    n_edges = edge_index.shape[1]
    d_out = W_res.shape[1]
    d_edge = edge_attr.shape[1]

    acc_rows = -(-(n_nodes + 1) // (NS * 8)) * (NS * 8)
    w_cat = jnp.concatenate([W_lin[:d_feat], W_res], axis=1)
    b_cat = jnp.concatenate([b_lin, b_res]).reshape(1, -1)
    proj, res = pl.pallas_call(
        _node_proj_kernel,
        out_shape=(jax.ShapeDtypeStruct((acc_rows, d_out), jnp.float32),
                   jax.ShapeDtypeStruct((n_nodes, d_out), jnp.float32)),
    )(feat, w_cat, b_cat)

    total_chunks = n_edges // CHUNK
    n_chunks = -(-total_chunks // NW)
    real8 = n_edges // 8
    w2bd = jnp.kron(jnp.eye(8, dtype=jnp.float32), W_lin[d_feat:])
    eproj = pl.pallas_call(
        _edge_proj_kernel,
        grid=(-(-real8 // EB),),
        in_specs=[pl.BlockSpec((EB, 8 * d_edge), lambda i: (i, 0)),
                  pl.BlockSpec((8 * d_edge, 8 * d_out), lambda i: (0, 0))],
        out_specs=pl.BlockSpec((EB, 8 * d_out), lambda i: (i, 0)),
        out_shape=jax.ShapeDtypeStruct((real8, 8 * d_out), jnp.float32),
    )(edge_attr.reshape(real8, 8 * d_edge), w2bd).reshape(n_edges, d_out)

    eidx = edge_index.astype(jnp.int32).reshape(2, -1, IDX_W)

    zeros = jnp.zeros((acc_rows, d_out), jnp.float32)
    parts = _make_sc_kernel(n_nodes, d_out, n_chunks, total_chunks, acc_rows)(
        proj, eproj, eidx, zeros)

    return pl.pallas_call(
        _final_kernel,
        out_shape=jax.ShapeDtypeStruct((n_nodes, d_out), jnp.float32),
    )(parts, res)
